# Initial kernel scaffold; baseline (speedup 1.0000x reference)
import jax
import jax.numpy as jnp
from jax.experimental import pallas as pl


def kernel(x, edge_index, W_gat, a_src, a_dst, b_gat, W_gcn, b_gcn, W_fc, b_fc):
    raise NotImplementedError("write your pallas kernel here")


# speedup vs baseline: 1.1691x; 1.1691x over previous
import jax
import jax.numpy as jnp
from jax.experimental import pallas as pl

_N = 10000
_E = 320000
_HID = 128
_HEADS = 8
_HDIM = 16
_LAYERS = 12
_STEPS = 5
_OUT = 128


def _fc_body(x_ref, w_ref, b_ref, o_ref):
    o_ref[...] = jnp.dot(x_ref[...], w_ref[...],
                         preferred_element_type=jnp.float32) + b_ref[...]


def _fc(x, W, b):
    n, d = x.shape
    out = W.shape[1]
    return pl.pallas_call(
        _fc_body,
        out_shape=jax.ShapeDtypeStruct((n, out), jnp.float32),
    )(x, W, b.reshape(1, out))


def _gcn(x, src, dst, W, b, n, dinv):
    h = x @ W
    coef = dinv[src] * dinv[dst]
    out = jax.ops.segment_sum(h[src] * coef[:, None], dst, num_segments=n)
    out = out + h * (dinv * dinv)[:, None]
    return out + b


def _gat(x, src, dst, W, a_s, a_d, b, n):
    loop = jnp.arange(n, dtype=src.dtype)
    s = jnp.concatenate([src, loop])
    d = jnp.concatenate([dst, loop])
    h = x @ W
    e = jax.nn.leaky_relu(h[s] @ a_s + h[d] @ a_d, negative_slope=0.2)
    emax = jax.ops.segment_max(e, d, num_segments=n)
    emax = jnp.where(jnp.isfinite(emax), emax, 0.0)
    ex = jnp.exp(e - emax[d])
    denom = jax.ops.segment_sum(ex, d, num_segments=n)
    alpha = ex / (denom[d] + 1e-16)
    out = jax.ops.segment_sum(h[s] * alpha[:, None], d, num_segments=n)
    return out + b


def kernel(x, edge_index, W_gat, a_src, a_dst, b_gat, W_gcn, b_gcn, W_fc, b_fc):
    n = x.shape[0]
    src = edge_index[0]
    dst = edge_index[1]
    h = x
    for _ in range(_STEPS):
        h = jax.ops.segment_sum(h[dst], src, num_segments=n)
    heads = [_gat(h, src, dst, W_gat[k], a_src[k], a_dst[k], b_gat[k], n)
             for k in range(_HEADS)]
    h = jnp.concatenate(heads, axis=1)
    deg = jax.ops.segment_sum(jnp.ones(_E, dtype=x.dtype), dst,
                              num_segments=n) + 1.0
    dinv = jax.lax.rsqrt(deg)
    init_res = h
    for i in range(_LAYERS):
        h = _gcn(h, src, dst, W_gcn[i], b_gcn[i], n, dinv)
        if i > 0:
            h = h + init_res
            init_res = h
        h = jax.nn.relu(h)
        if i < _LAYERS - 1:
            h = h + _gcn(h, src, dst, W_gcn[i - 1], b_gcn[i - 1], n, dinv)
    return _fc(h, W_fc, b_fc)


# speedup vs baseline: 1.8547x; 1.5864x over previous
import functools

import jax
import jax.numpy as jnp
from jax import lax
from jax.experimental import pallas as pl
from jax.experimental.pallas import tpu as pltpu
from jax.experimental.pallas import tpu_sc as plsc

_N = 10000
_E = 320000
_HID = 128
_HEADS = 8
_HDIM = 16
_LAYERS = 12
_STEPS = 5

_NC = 2
_NS = 16
_NW = _NC * _NS
_CH = 128
_CPW = -(-_E // (_NW * _CH))
_EPAD = _NW * _CPW * _CH
_NP = 10240
_RPT = _NP // _NS


def _spmv_body(table, pidx, out, pi_v, gi_v, si_v, rows_v, zero_v, acc, sem):
    c = lax.axis_index("c")
    s = lax.axis_index("s")
    wid = s * _NC + c
    pltpu.sync_copy(pidx.at[wid], pi_v)

    def _zrow(r, _):
        for j in range(_HID // 16):
            zero_v[r, pl.ds(j * 16, 16)] = jnp.zeros((16,), jnp.float32)
        return 0

    lax.fori_loop(0, _CH, _zrow, 0)
    for i in range(_RPT // _CH):
        pltpu.sync_copy(zero_v, acc.at[pl.ds(s * _RPT + i * _CH, _CH)])
    plsc.subcore_barrier()

    def _chunk(j, _):
        for i in range(_CH // 16):
            p = pi_v[j, pl.ds(i * 16, 16)]
            gi_v[pl.ds(i * 16, 16)] = lax.bitwise_and(p, 0xFFFF)
            si_v[pl.ds(i * 16, 16)] = lax.shift_right_logical(p, 16)
        pltpu.async_copy(table.at[gi_v], rows_v, sem).wait()
        pltpu.sync_copy(rows_v, acc.at[si_v], add=True)
        return 0

    lax.fori_loop(0, _CPW, _chunk, 0)
    plsc.subcore_barrier()
    pltpu.sync_copy(acc.at[pl.ds(s * _RPT, _RPT)],
                    out.at[c, pl.ds(s * _RPT, _RPT)])


def _spmv(table, pidx):
    return pl.kernel(
        _spmv_body,
        out_type=jax.ShapeDtypeStruct((_NC, _NP, _HID), jnp.float32),
        mesh=plsc.VectorSubcoreMesh(core_axis_name="c", subcore_axis_name="s"),
        scratch_types=[
            pltpu.VMEM((_CPW, _CH), jnp.int32),
            pltpu.VMEM((_CH,), jnp.int32),
            pltpu.VMEM((_CH,), jnp.int32),
            pltpu.VMEM((_CH, _HID), jnp.float32),
            pltpu.VMEM((_CH, _HID), jnp.float32),
            pltpu.VMEM_SHARED((_NP, _HID), jnp.float32),
            pltpu.SemaphoreType.DMA,
        ],
    )(table, pidx)


def _mk_idx(gidx, sidx):
    pad = _EPAD - _E
    packed = jnp.concatenate([
        gidx + (sidx << 16),
        jnp.full((pad,), _N << 16, jnp.int32)]).reshape(_NW, _CPW, _CH)
    return packed


_BLK = 512
_GRID = _NP // _BLK


def _add2_body(a_ref, b_ref, o_ref):
    o_ref[...] = a_ref[...] + b_ref[...]


def _tc_add2(a, b):
    return pl.pallas_call(
        _add2_body,
        out_shape=jax.ShapeDtypeStruct((_NP, _HID), jnp.float32),
        grid=(_GRID,),
        in_specs=[pl.BlockSpec((_BLK, _HID), lambda i: (i, 0))] * 2,
        out_specs=pl.BlockSpec((_BLK, _HID), lambda i: (i, 0)),
    )(a, b)


def _fc_body(x_ref, w_ref, b_ref, o_ref):
    o_ref[...] = jnp.dot(jnp.maximum(x_ref[...], 0.0), w_ref[...],
                         preferred_element_type=jnp.float32) + b_ref[...]


def _fc(x, W, b):
    n, _ = x.shape
    out = W.shape[1]
    return pl.pallas_call(
        _fc_body,
        out_shape=jax.ShapeDtypeStruct((n, out), jnp.float32),
    )(x, W, b.reshape(1, out))


def _mm_body(relu_in, x_ref, w_ref, s_ref, o_ref):
    x = x_ref[...]
    if relu_in:
        x = jnp.maximum(x, 0.0)
    o_ref[...] = s_ref[...] * jnp.dot(x, w_ref[...],
                                      preferred_element_type=jnp.float32)


def _tc_mm_scale(x, W, scale, relu_in=False):
    return pl.pallas_call(
        functools.partial(_mm_body, relu_in),
        out_shape=jax.ShapeDtypeStruct((_NP, _HID), jnp.float32),
        grid=(_GRID,),
        in_specs=[
            pl.BlockSpec((_BLK, _HID), lambda i: (i, 0)),
            pl.BlockSpec((_HID, _HID), lambda i: (0, 0)),
            pl.BlockSpec((_BLK, _HID), lambda i: (i, 0)),
        ],
        out_specs=pl.BlockSpec((_BLK, _HID), lambda i: (i, 0)),
    )(x, W, scale)


def _post_body(r_mode, p0_ref, p1_ref, u_ref, dv_ref, b_ref, r_ref, o_ref):
    g = dv_ref[...] * (p0_ref[...] + p1_ref[...] + u_ref[...]) + b_ref[...]
    if r_mode == "add":
        g = g + r_ref[...]
    elif r_mode == "relu":
        g = g + jnp.maximum(r_ref[...], 0.0)
    o_ref[...] = g


def _tc_post(p, u, dinvb, b, r, r_mode):
    return pl.pallas_call(
        functools.partial(_post_body, r_mode),
        out_shape=jax.ShapeDtypeStruct((_NP, _HID), jnp.float32),
        grid=(_GRID,),
        in_specs=[
            pl.BlockSpec((_BLK, _HID), lambda i: (i, 0)),
            pl.BlockSpec((_BLK, _HID), lambda i: (i, 0)),
            pl.BlockSpec((_BLK, _HID), lambda i: (i, 0)),
            pl.BlockSpec((_BLK, _HID), lambda i: (i, 0)),
            pl.BlockSpec((1, _HID), lambda i: (0, 0)),
            pl.BlockSpec((_BLK, _HID), lambda i: (i, 0)),
        ],
        out_specs=pl.BlockSpec((_BLK, _HID), lambda i: (i, 0)),
    )(p[0], p[1], u, dinvb, b.reshape(1, _HID), r)


def _dinv_body(p0_ref, p1_ref, o_ref):
    o_ref[...] = lax.rsqrt(p0_ref[...] + p1_ref[...] + 1.0)


def _tc_dinv(p):
    return pl.pallas_call(
        _dinv_body,
        out_shape=jax.ShapeDtypeStruct((_NP, _HID), jnp.float32),
        grid=(_GRID,),
        in_specs=[pl.BlockSpec((_BLK, _HID), lambda i: (i, 0))] * 2,
        out_specs=pl.BlockSpec((_BLK, _HID), lambda i: (i, 0)),
    )(p[0], p[1])


def _gat(x, src, dst, W, a_s, a_d, b, n):
    loop = jnp.arange(n, dtype=src.dtype)
    s = jnp.concatenate([src, loop])
    d = jnp.concatenate([dst, loop])
    h = x @ W
    e = jax.nn.leaky_relu(h[s] @ a_s + h[d] @ a_d, negative_slope=0.2)
    emax = jax.ops.segment_max(e, d, num_segments=n)
    emax = jnp.where(jnp.isfinite(emax), emax, 0.0)
    ex = jnp.exp(e - emax[d])
    denom = jax.ops.segment_sum(ex, d, num_segments=n)
    alpha = ex / (denom[d] + 1e-16)
    out = jax.ops.segment_sum(h[s] * alpha[:, None], d, num_segments=n)
    return out + b


def kernel(x, edge_index, W_gat, a_src, a_dst, b_gat, W_gcn, b_gcn, W_fc, b_fc):
    n = x.shape[0]
    src = edge_index[0]
    dst = edge_index[1]

    rw_idx = _mk_idx(dst, src)
    hp = jnp.pad(x, ((0, _NP - _N), (0, 0)))
    for _ in range(_STEPS):
        p = _spmv(hp, rw_idx)
        hp = _tc_add2(p[0], p[1])
    h = hp[:_N]

    heads = [_gat(h, src, dst, W_gat[k], a_src[k], a_dst[k], b_gat[k], n)
             for k in range(_HEADS)]
    hg = jnp.pad(jnp.concatenate(heads, axis=1), ((0, _NP - _N), (0, 0)))

    gcn_idx = _mk_idx(src, dst)
    ones = jnp.ones((_NP, _HID), jnp.float32)
    dinvb = _tc_dinv(_spmv(ones, gcn_idx))

    init_res = hg
    t = hg
    for i in range(_LAYERS):
        src_h = hg if i == 0 else h_cur
        uA = _tc_mm_scale(src_h, W_gcn[i], dinvb, relu_in=False)
        pA = _spmv(uA, gcn_idx)
        t = _tc_post(pA, uA, dinvb, b_gcn[i], init_res,
                     "add" if i > 0 else "none")
        if i > 0:
            init_res = t
        if i < _LAYERS - 1:
            uB = _tc_mm_scale(t, W_gcn[i - 1], dinvb, relu_in=True)
            pB = _spmv(uB, gcn_idx)
            h_cur = _tc_post(pB, uB, dinvb, b_gcn[i - 1], t, "relu")
    return _fc(t[:_N], W_fc, b_fc)
